# Initial kernel scaffold; baseline (speedup 1.0000x reference)
import jax
import jax.numpy as jnp
from jax.experimental import pallas as pl


def kernel(seed_xyz, seed_features, w_g1, b_g1, g_gamma, g_beta, g_mean, g_var, w_g2, b_g2, w1, b1, bn1_gamma, bn1_beta, bn1_mean, bn1_var, w2, b2):
    raise NotImplementedError("write your pallas kernel here")


# speedup vs baseline: 10.0320x; 10.0320x over previous
import functools

import jax
import jax.numpy as jnp
import numpy as np
from jax.experimental import pallas as pl

B, N, C = 4, 20000, 256
NS = 1024
RB, CB = 8, 2500
NBLK = N // CB

_BIG = 2 ** 30


def _grasp_views(n=300):
    phi = (np.sqrt(5) - 1) / 2
    i = np.arange(n)
    zi = (2 * i + 1) / n - 1
    xi = np.sqrt(np.maximum(1 - zi ** 2, 0.0)) * np.cos(2 * i * np.pi * phi)
    yi = np.sqrt(np.maximum(1 - zi ** 2, 0.0)) * np.sin(2 * i * np.pi * phi)
    return np.stack([xi, yi, zi], axis=1).astype(np.float32)


_TMPL = jnp.asarray(_grasp_views(300))


def _scores_body(f_ref, w1_ref, b1_ref, ga_ref, be_ref, mu_ref, va_ref,
                 w2_ref, b2_ref, g3_ref, mask_ref, g2r_ref):
    for n in range(NBLK):
        sl = pl.ds(n * CB, CB)
        f = f_ref[0, :, sl]
        x = jax.lax.dot_general(w1_ref[...], f, (((1,), (0,)), ((), ())),
                                preferred_element_type=jnp.float32)
        x = x + b1_ref[...]
        h = (x - mu_ref[...]) / jnp.sqrt(va_ref[...] + 1e-5) * ga_ref[...] + be_ref[...]
        h = jnp.maximum(h, 0.0)
        g = jax.lax.dot_general(w2_ref[...], h, (((1,), (0,)), ((), ())),
                                preferred_element_type=jnp.float32)
        g = g + b2_ref[...]
        g3_ref[0, :, sl] = g
        gr = g[2:3]
        m = (gr > 0.1) & (g[1:2] > g[0:1])
        mask_ref[0, n] = m.astype(jnp.float32)
        g2r_ref[0, n] = gr


def _scores(seed_features, w_g1, b_g1, g_gamma, g_beta, g_mean, g_var,
            w_g2, b_g2):
    col = lambda p: p.reshape(-1, 1)
    wspec = lambda s: pl.BlockSpec(s, lambda b: (0,) * len(s))
    return pl.pallas_call(
        _scores_body,
        grid=(B,),
        in_specs=[
            pl.BlockSpec((1, C, N), lambda b: (b, 0, 0)),
            wspec((C, C)), wspec((C, 1)), wspec((C, 1)), wspec((C, 1)),
            wspec((C, 1)), wspec((C, 1)), wspec((3, C)), wspec((3, 1)),
        ],
        out_specs=[
            pl.BlockSpec((1, 3, N), lambda b: (b, 0, 0)),
            pl.BlockSpec((1, RB, 1, CB), lambda b: (b, 0, 0, 0)),
            pl.BlockSpec((1, RB, 1, CB), lambda b: (b, 0, 0, 0)),
        ],
        out_shape=[
            jax.ShapeDtypeStruct((B, 3, N), jnp.float32),
            jax.ShapeDtypeStruct((B, RB, 1, CB), jnp.float32),
            jax.ShapeDtypeStruct((B, RB, 1, CB), jnp.float32),
        ],
    )(seed_features, w_g1, col(b_g1), col(g_gamma), col(g_beta),
      col(g_mean), col(g_var), w_g2, col(b_g2))


def _fps_body(xt_ref, mask_ref, g2r_ref,
              inds_ref, gx_ref, gy_ref, gz_ref, gp_ref):
    niota = (jax.lax.broadcasted_iota(jnp.int32, (RB, CB), 0) * CB
             + jax.lax.broadcasted_iota(jnp.int32, (RB, CB), 1))
    kiota = (jax.lax.broadcasted_iota(jnp.int32, (8, 128), 0) * 128
             + jax.lax.broadcasted_iota(jnp.int32, (8, 128), 1))
    ninf = jnp.float32(-jnp.inf)

    xs, ys, zs, ms, gs = [], [], [], [], []
    carry = []
    for b in range(B):
        xs.append(xt_ref[0, b])
        ys.append(xt_ref[1, b])
        zs.append(xt_ref[2, b])
        ms.append(mask_ref[b, :, 0, :] > 0.0)
        gs.append(g2r_ref[b, :, 0, :])
        cand = jnp.where(ms[b], niota, _BIG)
        first = jnp.min(cand)
        first = jnp.where(first == _BIG, 0, first)
        d0 = jnp.full((RB, CB), 1e10, dtype=jnp.float32)
        zvec_f = jnp.zeros((8, 128), jnp.float32)
        zvec_i = jnp.zeros((8, 128), jnp.int32)
        carry.extend([d0, first, zvec_i, zvec_f, zvec_f, zvec_f, zvec_f])

    def step(k, carry):
        carry = list(carry)
        for b in range(B):
            d, last, iv, vx, vy, vz, vg = carry[7 * b:7 * b + 7]
            sel = kiota == k
            match = niota == last
            lx = jnp.max(jnp.where(match, xs[b], ninf))
            ly = jnp.max(jnp.where(match, ys[b], ninf))
            lz = jnp.max(jnp.where(match, zs[b], ninf))
            lg = jnp.max(jnp.where(match, gs[b], ninf))
            iv = jnp.where(sel, last, iv)
            vx = jnp.where(sel, lx, vx)
            vy = jnp.where(sel, ly, vy)
            vz = jnp.where(sel, lz, vz)
            vg = jnp.where(sel, lg, vg)
            dx = xs[b] - lx
            dy = ys[b] - ly
            dz = zs[b] - lz
            cur = (dx * dx + dy * dy) + dz * dz
            d = jnp.minimum(d, cur)
            masked = jnp.where(ms[b], d, -1.0)
            m = jnp.max(masked)
            nxt = jnp.min(jnp.where(masked == m, niota, _BIG))
            carry[7 * b:7 * b + 7] = [d, nxt, iv, vx, vy, vz, vg]
        return tuple(carry)

    carry = jax.lax.fori_loop(0, NS, step, tuple(carry))
    for b in range(B):
        _, _, iv, vx, vy, vz, vg = carry[7 * b:7 * b + 7]
        inds_ref[b] = iv
        gx_ref[b] = vx
        gy_ref[b] = vy
        gz_ref[b] = vz
        gp_ref[b] = vg


def _fps(xt, mask, g2r):
    shp = jax.ShapeDtypeStruct((B, 8, 128), jnp.float32)
    return pl.pallas_call(
        _fps_body,
        out_shape=[jax.ShapeDtypeStruct((B, 8, 128), jnp.int32),
                   shp, shp, shp, shp],
    )(xt, mask, g2r)


def _gather_body(inds_ref, f_ref, o_ref):
    iv = inds_ref[0]
    acc = None
    for n in range(NBLK):
        rowi = (jax.lax.broadcasted_iota(jnp.int32, (CB, NS), 0) + n * CB)
        p = (rowi == iv).astype(jnp.float32)
        g = jax.lax.dot_general(f_ref[0, :, pl.ds(n * CB, CB)], p,
                                (((1,), (0,)), ((), ())),
                                preferred_element_type=jnp.float32)
        acc = g if acc is None else acc + g
    o_ref[0] = acc


def _gather(seed_features, inds):
    return pl.pallas_call(
        _gather_body,
        grid=(B,),
        in_specs=[
            pl.BlockSpec((1, 1, NS), lambda b: (b, 0, 0)),
            pl.BlockSpec((1, C, N), lambda b: (b, 0, 0)),
        ],
        out_specs=pl.BlockSpec((1, C, NS), lambda b: (b, 0, 0)),
        out_shape=jax.ShapeDtypeStruct((B, C, NS), jnp.float32),
    )(inds.reshape(B, 1, NS), seed_features)


def _head_body(gf_ref, w1_ref, b1_ref, ga_ref, be_ref, mu_ref, va_ref,
               w2_ref, b2_ref, tmpl_ref, v_ref, tvi_ref, rot_ref):
    gf = gf_ref[0]
    t = jax.lax.dot_general(w1_ref[...], gf, (((1,), (0,)), ((), ())),
                            preferred_element_type=jnp.float32)
    t = t + b1_ref[...]
    t = (t - mu_ref[...]) / jnp.sqrt(va_ref[...] + 1e-5) * ga_ref[...] + be_ref[...]
    t = jnp.maximum(t, 0.0)
    v = jax.lax.dot_general(w2_ref[...], t, (((1,), (0,)), ((), ())),
                            preferred_element_type=jnp.float32)
    v = v + b2_ref[...]
    v_ref[0] = v

    vx, vy, vz = v[0:1], v[1:2], v[2:3]
    tx, ty, tz = (tmpl_ref[:, 0:1], tmpl_ref[:, 1:2], tmpl_ref[:, 2:3])
    na = jnp.maximum(jnp.sqrt((tx * tx + ty * ty) + tz * tz), 1e-8)
    nb = jnp.maximum(jnp.sqrt((vx * vx + vy * vy) + vz * vz), 1e-8)
    s = ((tx * vx + ty * vy) + tz * vz) / (na * nb)
    smax = jnp.max(s, axis=0, keepdims=True)
    tiota = jax.lax.broadcasted_iota(jnp.int32, s.shape, 0)
    tvi_ref[0] = jnp.min(jnp.where(s == smax, tiota, _BIG), axis=0,
                         keepdims=True)

    ax0, ax1, ax2 = -vx, -vy, -vz
    ay0, ay1, ay2 = -ax1, ax0, jnp.zeros_like(ax0)
    ny = jnp.sqrt((ay0 * ay0 + ay1 * ay1) + ay2 * ay2)
    y_zero = ny == 0.0
    ay0 = jnp.where(y_zero, 0.0, ay0)
    ay1 = jnp.where(y_zero, 1.0, ay1)
    ay2 = jnp.where(y_zero, 0.0, ay2)
    nx = jnp.sqrt((ax0 * ax0 + ax1 * ax1) + ax2 * ax2)
    ax0, ax1, ax2 = ax0 / nx, ax1 / nx, ax2 / nx
    ny = jnp.sqrt((ay0 * ay0 + ay1 * ay1) + ay2 * ay2)
    ay0, ay1, ay2 = ay0 / ny, ay1 / ny, ay2 / ny
    az0 = ax1 * ay2 - ax2 * ay1
    az1 = ax2 * ay0 - ax0 * ay2
    az2 = ax0 * ay1 - ax1 * ay0
    rot = jnp.concatenate([ax0, ay0, az0, ax1, ay1, az1, ax2, ay2, az2],
                          axis=0)
    rot_ref[0] = rot


def _head(gf, w1, b1, bn1_gamma, bn1_beta, bn1_mean, bn1_var, w2, b2):
    col = lambda p: p.reshape(-1, 1)
    wspec = lambda s: pl.BlockSpec(s, lambda b: (0,) * len(s))
    return pl.pallas_call(
        _head_body,
        grid=(B,),
        in_specs=[
            pl.BlockSpec((1, C, NS), lambda b: (b, 0, 0)),
            wspec((C, C)), wspec((C, 1)), wspec((C, 1)), wspec((C, 1)),
            wspec((C, 1)), wspec((C, 1)), wspec((3, C)), wspec((3, 1)),
            wspec((300, 3)),
        ],
        out_specs=[
            pl.BlockSpec((1, 3, NS), lambda b: (b, 0, 0)),
            pl.BlockSpec((1, 1, NS), lambda b: (b, 0, 0)),
            pl.BlockSpec((1, 9, NS), lambda b: (b, 0, 0)),
        ],
        out_shape=[
            jax.ShapeDtypeStruct((B, 3, NS), jnp.float32),
            jax.ShapeDtypeStruct((B, 1, NS), jnp.int32),
            jax.ShapeDtypeStruct((B, 9, NS), jnp.float32),
        ],
    )(gf, w1, col(b1), col(bn1_gamma), col(bn1_beta), col(bn1_mean),
      col(bn1_var), w2, col(b2), _TMPL)


@jax.jit
def kernel(seed_xyz, seed_features, w_g1, b_g1, g_gamma, g_beta, g_mean,
           g_var, w_g2, b_g2, w1, b1, bn1_gamma, bn1_beta, bn1_mean,
           bn1_var, w2, b2):
    g3, mask, g2r = _scores(seed_features, w_g1, b_g1, g_gamma, g_beta,
                            g_mean, g_var, w_g2, b_g2)
    objectness_score = g3[:, :2]
    graspness_score = g3[:, 2]

    xt = jnp.transpose(seed_xyz, (2, 0, 1)).reshape(3, B, RB, CB)
    inds8, gx, gy, gz, gp = _fps(xt, mask, g2r)
    graspable_inds = inds8.reshape(B, NS)
    graspable_xyz = jnp.stack(
        [gx.reshape(B, NS), gy.reshape(B, NS), gz.reshape(B, NS)], axis=-1)
    fp2_graspness = gp.reshape(B, NS)

    graspable_features = _gather(seed_features, graspable_inds)

    v, tvi, rot = _head(graspable_features, w1, b1, bn1_gamma, bn1_beta,
                        bn1_mean, bn1_var, w2, b2)
    vp_xyz = jnp.transpose(v, (0, 2, 1))
    top_view_inds = tvi.reshape(B, NS)
    vp_rot = jnp.transpose(rot, (0, 2, 1)).reshape(B, NS, 3, 3)

    return (objectness_score, graspness_score, graspable_xyz,
            graspable_inds, graspable_features, fp2_graspness, vp_xyz,
            top_view_inds, vp_rot)


# speedup vs baseline: 11.2094x; 1.1174x over previous
import functools

import jax
import jax.numpy as jnp
import numpy as np
from jax.experimental import pallas as pl
from jax.experimental.pallas import tpu as pltpu

B, N, C = 4, 20000, 256
NS = 1024
RB, CB = 8, 2500
NBLK = N // CB

_BIG = 2 ** 30


def _grasp_views(n=300):
    phi = (np.sqrt(5) - 1) / 2
    i = np.arange(n)
    zi = (2 * i + 1) / n - 1
    xi = np.sqrt(np.maximum(1 - zi ** 2, 0.0)) * np.cos(2 * i * np.pi * phi)
    yi = np.sqrt(np.maximum(1 - zi ** 2, 0.0)) * np.sin(2 * i * np.pi * phi)
    return np.stack([xi, yi, zi], axis=1).astype(np.float32)


_TMPL = _grasp_views(300)


def _scores_body(f_ref, w1_ref, b1_ref, ga_ref, be_ref, mu_ref, va_ref,
                 w2_ref, b2_ref, g3_ref, mask_ref, g2r_ref):
    for n in range(NBLK):
        sl = pl.ds(n * CB, CB)
        f = f_ref[0, :, sl]
        x = jax.lax.dot_general(w1_ref[...], f, (((1,), (0,)), ((), ())),
                                preferred_element_type=jnp.float32)
        x = x + b1_ref[...]
        h = (x - mu_ref[...]) / jnp.sqrt(va_ref[...] + 1e-5) * ga_ref[...] + be_ref[...]
        h = jnp.maximum(h, 0.0)
        g = jax.lax.dot_general(w2_ref[...], h, (((1,), (0,)), ((), ())),
                                preferred_element_type=jnp.float32)
        g = g + b2_ref[...]
        g3_ref[0, :, sl] = g
        gr = g[2:3]
        m = (gr > 0.1) & (g[1:2] > g[0:1])
        mask_ref[0, n] = m.astype(jnp.float32)
        g2r_ref[0, n] = gr


def _scores(seed_features, w_g1, b_g1, g_gamma, g_beta, g_mean, g_var,
            w_g2, b_g2):
    col = lambda p: p.reshape(-1, 1)
    wspec = lambda s: pl.BlockSpec(s, lambda b: (0,) * len(s))
    return pl.pallas_call(
        _scores_body,
        grid=(B,),
        in_specs=[
            pl.BlockSpec((1, C, N), lambda b: (b, 0, 0)),
            wspec((C, C)), wspec((C, 1)), wspec((C, 1)), wspec((C, 1)),
            wspec((C, 1)), wspec((C, 1)), wspec((3, C)), wspec((3, 1)),
        ],
        out_specs=[
            pl.BlockSpec((1, 3, N), lambda b: (b, 0, 0)),
            pl.BlockSpec((1, RB, 1, CB), lambda b: (b, 0, 0, 0)),
            pl.BlockSpec((1, RB, 1, CB), lambda b: (b, 0, 0, 0)),
        ],
        out_shape=[
            jax.ShapeDtypeStruct((B, 3, N), jnp.float32),
            jax.ShapeDtypeStruct((B, RB, 1, CB), jnp.float32),
            jax.ShapeDtypeStruct((B, RB, 1, CB), jnp.float32),
        ],
    )(seed_features, w_g1, col(b_g1), col(g_gamma), col(g_beta),
      col(g_mean), col(g_var), w_g2, col(b_g2))


NR = 160
NPAD = NR * 128
NCH = NR // 8


def _fps_body(xt_ref, mask_ref, g2r_ref,
              inds_ref, gx_ref, gy_ref, gz_ref, gp_ref, d_ref):
    kiota = (jax.lax.broadcasted_iota(jnp.int32, (8, 128), 0) * 128
             + jax.lax.broadcasted_iota(jnp.int32, (8, 128), 1))
    liota = jax.lax.broadcasted_iota(jnp.int32, (1, 128), 1)
    ninf = jnp.float32(-jnp.inf)

    firsts = []
    for b in range(B):
        pid = jnp.full((8, 128), _BIG, jnp.int32)
        for c in range(NCH):
            mc = mask_ref[b, 8 * c:8 * c + 8, :] > 0.0
            d_ref[b, 8 * c:8 * c + 8, :] = jnp.where(mc, 1e10, -1.0)
            pid = jnp.minimum(pid, jnp.where(mc, kiota + 1024 * c, _BIG))
        first = jnp.min(pid)
        firsts.append(jnp.where(first == _BIG, 0, first))

    def extract(rv, l):
        return jnp.max(jnp.where(liota == l, rv, ninf))

    zf = jnp.zeros((8, 128), jnp.float32)
    zi = jnp.zeros((8, 128), jnp.int32)
    carry = []
    for b in range(B):
        carry.extend([firsts[b], zi, zf, zf, zf, zf])

    def step(k, carry):
        carry = list(carry)
        for b in range(B):
            last, iv, vx, vy, vz, vg = carry[6 * b:6 * b + 6]
            r = jax.lax.shift_right_logical(last, 7)
            l = last & 127
            lx = extract(xt_ref[0, b, pl.ds(r, 1), :], l)
            ly = extract(xt_ref[1, b, pl.ds(r, 1), :], l)
            lz = extract(xt_ref[2, b, pl.ds(r, 1), :], l)
            lg = extract(g2r_ref[b, pl.ds(r, 1), :], l)
            sel = kiota == k
            iv = jnp.where(sel, last, iv)
            vx = jnp.where(sel, lx, vx)
            vy = jnp.where(sel, ly, vy)
            vz = jnp.where(sel, lz, vz)
            vg = jnp.where(sel, lg, vg)
            pmax = jnp.full((8, 128), ninf)
            for c in range(NCH):
                rows = slice(8 * c, 8 * c + 8)
                dx = xt_ref[0, b, rows, :] - lx
                dy = xt_ref[1, b, rows, :] - ly
                dz = xt_ref[2, b, rows, :] - lz
                cur = (dx * dx + dy * dy) + dz * dz
                dc = jnp.minimum(d_ref[b, rows, :], cur)
                d_ref[b, rows, :] = dc
                pmax = jnp.maximum(pmax, dc)
            m = jnp.max(pmax)
            pid = jnp.full((8, 128), _BIG, jnp.int32)
            for c in range(NCH):
                dc = d_ref[b, 8 * c:8 * c + 8, :]
                pid = jnp.minimum(
                    pid, jnp.where(dc == m, kiota + 1024 * c, _BIG))
            nxt = jnp.min(pid)
            carry[6 * b:6 * b + 6] = [nxt, iv, vx, vy, vz, vg]
        return tuple(carry)

    carry = jax.lax.fori_loop(0, NS, step, tuple(carry))
    for b in range(B):
        _, iv, vx, vy, vz, vg = carry[6 * b:6 * b + 6]
        inds_ref[b] = iv
        gx_ref[b] = vx
        gy_ref[b] = vy
        gz_ref[b] = vz
        gp_ref[b] = vg


def _fps(xt, mask, g2r):
    shp = jax.ShapeDtypeStruct((B, 8, 128), jnp.float32)
    return pl.pallas_call(
        _fps_body,
        out_shape=[jax.ShapeDtypeStruct((B, 8, 128), jnp.int32),
                   shp, shp, shp, shp],
        scratch_shapes=[pltpu.VMEM((B, NR, 128), jnp.float32)],
    )(xt, mask, g2r)


def _gather_body(inds_ref, f_ref, o_ref):
    iv = inds_ref[0]
    acc = None
    for n in range(NBLK):
        rowi = (jax.lax.broadcasted_iota(jnp.int32, (CB, NS), 0) + n * CB)
        p = (rowi == iv).astype(jnp.float32)
        g = jax.lax.dot_general(f_ref[0, :, pl.ds(n * CB, CB)], p,
                                (((1,), (0,)), ((), ())),
                                preferred_element_type=jnp.float32)
        acc = g if acc is None else acc + g
    o_ref[0] = acc


def _gather(seed_features, inds):
    return pl.pallas_call(
        _gather_body,
        grid=(B,),
        in_specs=[
            pl.BlockSpec((1, 1, NS), lambda b: (b, 0, 0)),
            pl.BlockSpec((1, C, N), lambda b: (b, 0, 0)),
        ],
        out_specs=pl.BlockSpec((1, C, NS), lambda b: (b, 0, 0)),
        out_shape=jax.ShapeDtypeStruct((B, C, NS), jnp.float32),
    )(inds.reshape(B, 1, NS), seed_features)


def _head_body(gf_ref, w1_ref, b1_ref, ga_ref, be_ref, mu_ref, va_ref,
               w2_ref, b2_ref, tmpl_ref, v_ref, tvi_ref, rot_ref):
    gf = gf_ref[0]
    t = jax.lax.dot_general(w1_ref[...], gf, (((1,), (0,)), ((), ())),
                            preferred_element_type=jnp.float32)
    t = t + b1_ref[...]
    t = (t - mu_ref[...]) / jnp.sqrt(va_ref[...] + 1e-5) * ga_ref[...] + be_ref[...]
    t = jnp.maximum(t, 0.0)
    v = jax.lax.dot_general(w2_ref[...], t, (((1,), (0,)), ((), ())),
                            preferred_element_type=jnp.float32)
    v = v + b2_ref[...]
    v_ref[0] = v

    vx, vy, vz = v[0:1], v[1:2], v[2:3]
    tx, ty, tz = (tmpl_ref[:, 0:1], tmpl_ref[:, 1:2], tmpl_ref[:, 2:3])
    na = jnp.maximum(jnp.sqrt((tx * tx + ty * ty) + tz * tz), 1e-8)
    nb = jnp.maximum(jnp.sqrt((vx * vx + vy * vy) + vz * vz), 1e-8)
    s = ((tx * vx + ty * vy) + tz * vz) / (na * nb)
    smax = jnp.max(s, axis=0, keepdims=True)
    tiota = jax.lax.broadcasted_iota(jnp.int32, s.shape, 0)
    tvi_ref[0] = jnp.min(jnp.where(s == smax, tiota, _BIG), axis=0,
                         keepdims=True)

    ax0, ax1, ax2 = -vx, -vy, -vz
    ay0, ay1, ay2 = -ax1, ax0, jnp.zeros_like(ax0)
    ny = jnp.sqrt((ay0 * ay0 + ay1 * ay1) + ay2 * ay2)
    y_zero = ny == 0.0
    ay0 = jnp.where(y_zero, 0.0, ay0)
    ay1 = jnp.where(y_zero, 1.0, ay1)
    ay2 = jnp.where(y_zero, 0.0, ay2)
    nx = jnp.sqrt((ax0 * ax0 + ax1 * ax1) + ax2 * ax2)
    ax0, ax1, ax2 = ax0 / nx, ax1 / nx, ax2 / nx
    ny = jnp.sqrt((ay0 * ay0 + ay1 * ay1) + ay2 * ay2)
    ay0, ay1, ay2 = ay0 / ny, ay1 / ny, ay2 / ny
    az0 = ax1 * ay2 - ax2 * ay1
    az1 = ax2 * ay0 - ax0 * ay2
    az2 = ax0 * ay1 - ax1 * ay0
    rot = jnp.concatenate([ax0, ay0, az0, ax1, ay1, az1, ax2, ay2, az2],
                          axis=0)
    rot_ref[0] = rot


def _head(gf, w1, b1, bn1_gamma, bn1_beta, bn1_mean, bn1_var, w2, b2):
    col = lambda p: p.reshape(-1, 1)
    wspec = lambda s: pl.BlockSpec(s, lambda b: (0,) * len(s))
    return pl.pallas_call(
        _head_body,
        grid=(B,),
        in_specs=[
            pl.BlockSpec((1, C, NS), lambda b: (b, 0, 0)),
            wspec((C, C)), wspec((C, 1)), wspec((C, 1)), wspec((C, 1)),
            wspec((C, 1)), wspec((C, 1)), wspec((3, C)), wspec((3, 1)),
            wspec((300, 3)),
        ],
        out_specs=[
            pl.BlockSpec((1, 3, NS), lambda b: (b, 0, 0)),
            pl.BlockSpec((1, 1, NS), lambda b: (b, 0, 0)),
            pl.BlockSpec((1, 9, NS), lambda b: (b, 0, 0)),
        ],
        out_shape=[
            jax.ShapeDtypeStruct((B, 3, NS), jnp.float32),
            jax.ShapeDtypeStruct((B, 1, NS), jnp.int32),
            jax.ShapeDtypeStruct((B, 9, NS), jnp.float32),
        ],
    )(gf, w1, col(b1), col(bn1_gamma), col(bn1_beta), col(bn1_mean),
      col(bn1_var), w2, col(b2), _TMPL)


@jax.jit
def kernel(seed_xyz, seed_features, w_g1, b_g1, g_gamma, g_beta, g_mean,
           g_var, w_g2, b_g2, w1, b1, bn1_gamma, bn1_beta, bn1_mean,
           bn1_var, w2, b2):
    g3, mask, g2r = _scores(seed_features, w_g1, b_g1, g_gamma, g_beta,
                            g_mean, g_var, w_g2, b_g2)
    objectness_score = g3[:, :2]
    graspness_score = g3[:, 2]

    pad = lambda a: jnp.pad(a.reshape(-1, N), ((0, 0), (0, NPAD - N))
                            ).reshape(-1, NR, 128)
    xt = pad(jnp.transpose(seed_xyz, (2, 0, 1))).reshape(3, B, NR, 128)
    inds8, gx, gy, gz, gp = _fps(xt, pad(mask), pad(g2r))
    graspable_inds = inds8.reshape(B, NS)
    graspable_xyz = jnp.stack(
        [gx.reshape(B, NS), gy.reshape(B, NS), gz.reshape(B, NS)], axis=-1)
    fp2_graspness = gp.reshape(B, NS)

    graspable_features = _gather(seed_features, graspable_inds)

    v, tvi, rot = _head(graspable_features, w1, b1, bn1_gamma, bn1_beta,
                        bn1_mean, bn1_var, w2, b2)
    vp_xyz = jnp.transpose(v, (0, 2, 1))
    top_view_inds = tvi.reshape(B, NS)
    vp_rot = jnp.transpose(rot, (0, 2, 1)).reshape(B, NS, 3, 3)

    return (objectness_score, graspness_score, graspable_xyz,
            graspable_inds, graspable_features, fp2_graspness, vp_xyz,
            top_view_inds, vp_rot)


# speedup vs baseline: 27.2581x; 2.4317x over previous
import functools

import jax
import jax.numpy as jnp
import numpy as np
from jax.experimental import pallas as pl
from jax.experimental.pallas import tpu as pltpu

B, N, C = 4, 20000, 256
NS = 1024
RB, CB = 8, 2500
NBLK = N // CB

_BIG = 2 ** 30


def _grasp_views(n=300):
    phi = (np.sqrt(5) - 1) / 2
    i = np.arange(n)
    zi = (2 * i + 1) / n - 1
    xi = np.sqrt(np.maximum(1 - zi ** 2, 0.0)) * np.cos(2 * i * np.pi * phi)
    yi = np.sqrt(np.maximum(1 - zi ** 2, 0.0)) * np.sin(2 * i * np.pi * phi)
    return np.stack([xi, yi, zi], axis=1).astype(np.float32)


_TMPL = _grasp_views(300)


def _scores_body(f_ref, w1_ref, b1_ref, ga_ref, be_ref, mu_ref, va_ref,
                 w2_ref, b2_ref, g3_ref, mask_ref, g2r_ref):
    for n in range(NBLK):
        sl = pl.ds(n * CB, CB)
        f = f_ref[0, :, sl]
        x = jax.lax.dot_general(w1_ref[...], f, (((1,), (0,)), ((), ())),
                                preferred_element_type=jnp.float32)
        x = x + b1_ref[...]
        h = (x - mu_ref[...]) / jnp.sqrt(va_ref[...] + 1e-5) * ga_ref[...] + be_ref[...]
        h = jnp.maximum(h, 0.0)
        g = jax.lax.dot_general(w2_ref[...], h, (((1,), (0,)), ((), ())),
                                preferred_element_type=jnp.float32)
        g = g + b2_ref[...]
        g3_ref[0, :, sl] = g
        gr = g[2:3]
        m = (gr > 0.1) & (g[1:2] > g[0:1])
        mask_ref[0, n] = m.astype(jnp.float32)
        g2r_ref[0, n] = gr


def _scores(seed_features, w_g1, b_g1, g_gamma, g_beta, g_mean, g_var,
            w_g2, b_g2):
    col = lambda p: p.reshape(-1, 1)
    wspec = lambda s: pl.BlockSpec(s, lambda b: (0,) * len(s))
    return pl.pallas_call(
        _scores_body,
        grid=(B,),
        in_specs=[
            pl.BlockSpec((1, C, N), lambda b: (b, 0, 0)),
            wspec((C, C)), wspec((C, 1)), wspec((C, 1)), wspec((C, 1)),
            wspec((C, 1)), wspec((C, 1)), wspec((3, C)), wspec((3, 1)),
        ],
        out_specs=[
            pl.BlockSpec((1, 3, N), lambda b: (b, 0, 0)),
            pl.BlockSpec((1, RB, 1, CB), lambda b: (b, 0, 0, 0)),
            pl.BlockSpec((1, RB, 1, CB), lambda b: (b, 0, 0, 0)),
        ],
        out_shape=[
            jax.ShapeDtypeStruct((B, 3, N), jnp.float32),
            jax.ShapeDtypeStruct((B, RB, 1, CB), jnp.float32),
            jax.ShapeDtypeStruct((B, RB, 1, CB), jnp.float32),
        ],
    )(seed_features, w_g1, col(b_g1), col(g_gamma), col(g_beta),
      col(g_mean), col(g_var), w_g2, col(b_g2))


NR = 160
NPAD = NR * 128
NCH = NR // 8


def _seg_reduce(x, op):
    r = x.reshape(B, 8, 128)
    r = op(r, axis=1, keepdims=True)
    r = op(r, axis=2, keepdims=True)
    return jnp.broadcast_to(r, (B, 8, 128)).reshape(B * 8, 128)


def _fps_body(x_ref, y_ref, z_ref, g_ref, m_ref,
              inds_ref, gx_ref, gy_ref, gz_ref, gp_ref, d_ref):
    kiota = (jax.lax.broadcasted_iota(jnp.int32, (B, 8, 128), 1) * 128
             + jax.lax.broadcasted_iota(jnp.int32, (B, 8, 128), 2)
             ).reshape(B * 8, 128)
    ninf = jnp.float32(-jnp.inf)

    pid = jnp.full((B * 8, 128), _BIG, jnp.int32)
    for c in range(NCH):
        mc = m_ref[c] > 0.0
        d_ref[c] = jnp.where(mc, 1e10, -1.0)
        pid = jnp.minimum(pid, jnp.where(mc, kiota + 1024 * c, _BIG))
    first = _seg_reduce(pid, jnp.min)
    last0 = jnp.where(first == _BIG, 0, first)

    zf = jnp.zeros((B * 8, 128), jnp.float32)
    zi = jnp.zeros((B * 8, 128), jnp.int32)

    def step(k, carry):
        last, iv, vx, vy, vz, vg = carry
        px = jnp.full((B * 8, 128), ninf)
        py, pz, pg = px, px, px
        for c in range(NCH):
            e = (kiota + 1024 * c) == last
            px = jnp.maximum(px, jnp.where(e, x_ref[c], ninf))
            py = jnp.maximum(py, jnp.where(e, y_ref[c], ninf))
            pz = jnp.maximum(pz, jnp.where(e, z_ref[c], ninf))
            pg = jnp.maximum(pg, jnp.where(e, g_ref[c], ninf))
        lx = _seg_reduce(px, jnp.max)
        ly = _seg_reduce(py, jnp.max)
        lz = _seg_reduce(pz, jnp.max)
        lg = _seg_reduce(pg, jnp.max)
        sel = kiota == k
        iv = jnp.where(sel, last, iv)
        vx = jnp.where(sel, lx, vx)
        vy = jnp.where(sel, ly, vy)
        vz = jnp.where(sel, lz, vz)
        vg = jnp.where(sel, lg, vg)
        pmax = jnp.full((B * 8, 128), ninf)
        for c in range(NCH):
            dx = x_ref[c] - lx
            dy = y_ref[c] - ly
            dz = z_ref[c] - lz
            cur = (dx * dx + dy * dy) + dz * dz
            dc = jnp.minimum(d_ref[c], cur)
            d_ref[c] = dc
            pmax = jnp.maximum(pmax, dc)
        mv = _seg_reduce(pmax, jnp.max)
        pid = jnp.full((B * 8, 128), _BIG, jnp.int32)
        for c in range(NCH):
            dc = d_ref[c]
            pid = jnp.minimum(
                pid, jnp.where(dc == mv, kiota + 1024 * c, _BIG))
        nxt = _seg_reduce(pid, jnp.min)
        return (nxt, iv, vx, vy, vz, vg)

    carry = jax.lax.fori_loop(0, NS, step, (last0, zi, zf, zf, zf, zf))
    _, iv, vx, vy, vz, vg = carry
    inds_ref[...] = iv.reshape(B, 8, 128)
    gx_ref[...] = vx.reshape(B, 8, 128)
    gy_ref[...] = vy.reshape(B, 8, 128)
    gz_ref[...] = vz.reshape(B, 8, 128)
    gp_ref[...] = vg.reshape(B, 8, 128)


def _fps(xc, yc, zc, gc, mc):
    shp = jax.ShapeDtypeStruct((B, 8, 128), jnp.float32)
    return pl.pallas_call(
        _fps_body,
        out_shape=[jax.ShapeDtypeStruct((B, 8, 128), jnp.int32),
                   shp, shp, shp, shp],
        scratch_shapes=[pltpu.VMEM((NCH, B * 8, 128), jnp.float32)],
    )(xc, yc, zc, gc, mc)


def _gather_body(inds_ref, f_ref, o_ref):
    iv = inds_ref[0]
    acc = None
    for n in range(NBLK):
        rowi = (jax.lax.broadcasted_iota(jnp.int32, (CB, NS), 0) + n * CB)
        p = (rowi == iv).astype(jnp.float32)
        g = jax.lax.dot_general(f_ref[0, :, pl.ds(n * CB, CB)], p,
                                (((1,), (0,)), ((), ())),
                                preferred_element_type=jnp.float32)
        acc = g if acc is None else acc + g
    o_ref[0] = acc


def _gather(seed_features, inds):
    return pl.pallas_call(
        _gather_body,
        grid=(B,),
        in_specs=[
            pl.BlockSpec((1, 1, NS), lambda b: (b, 0, 0)),
            pl.BlockSpec((1, C, N), lambda b: (b, 0, 0)),
        ],
        out_specs=pl.BlockSpec((1, C, NS), lambda b: (b, 0, 0)),
        out_shape=jax.ShapeDtypeStruct((B, C, NS), jnp.float32),
    )(inds.reshape(B, 1, NS), seed_features)


def _head_body(gf_ref, w1_ref, b1_ref, ga_ref, be_ref, mu_ref, va_ref,
               w2_ref, b2_ref, tmpl_ref, v_ref, tvi_ref, rot_ref):
    gf = gf_ref[0]
    t = jax.lax.dot_general(w1_ref[...], gf, (((1,), (0,)), ((), ())),
                            preferred_element_type=jnp.float32)
    t = t + b1_ref[...]
    t = (t - mu_ref[...]) / jnp.sqrt(va_ref[...] + 1e-5) * ga_ref[...] + be_ref[...]
    t = jnp.maximum(t, 0.0)
    v = jax.lax.dot_general(w2_ref[...], t, (((1,), (0,)), ((), ())),
                            preferred_element_type=jnp.float32)
    v = v + b2_ref[...]
    v_ref[0] = v

    vx, vy, vz = v[0:1], v[1:2], v[2:3]
    tx, ty, tz = (tmpl_ref[:, 0:1], tmpl_ref[:, 1:2], tmpl_ref[:, 2:3])
    na = jnp.maximum(jnp.sqrt((tx * tx + ty * ty) + tz * tz), 1e-8)
    nb = jnp.maximum(jnp.sqrt((vx * vx + vy * vy) + vz * vz), 1e-8)
    s = ((tx * vx + ty * vy) + tz * vz) / (na * nb)
    smax = jnp.max(s, axis=0, keepdims=True)
    tiota = jax.lax.broadcasted_iota(jnp.int32, s.shape, 0)
    tvi_ref[0] = jnp.min(jnp.where(s == smax, tiota, _BIG), axis=0,
                         keepdims=True)

    ax0, ax1, ax2 = -vx, -vy, -vz
    ay0, ay1, ay2 = -ax1, ax0, jnp.zeros_like(ax0)
    ny = jnp.sqrt((ay0 * ay0 + ay1 * ay1) + ay2 * ay2)
    y_zero = ny == 0.0
    ay0 = jnp.where(y_zero, 0.0, ay0)
    ay1 = jnp.where(y_zero, 1.0, ay1)
    ay2 = jnp.where(y_zero, 0.0, ay2)
    nx = jnp.sqrt((ax0 * ax0 + ax1 * ax1) + ax2 * ax2)
    ax0, ax1, ax2 = ax0 / nx, ax1 / nx, ax2 / nx
    ny = jnp.sqrt((ay0 * ay0 + ay1 * ay1) + ay2 * ay2)
    ay0, ay1, ay2 = ay0 / ny, ay1 / ny, ay2 / ny
    az0 = ax1 * ay2 - ax2 * ay1
    az1 = ax2 * ay0 - ax0 * ay2
    az2 = ax0 * ay1 - ax1 * ay0
    rot = jnp.concatenate([ax0, ay0, az0, ax1, ay1, az1, ax2, ay2, az2],
                          axis=0)
    rot_ref[0] = rot


def _head(gf, w1, b1, bn1_gamma, bn1_beta, bn1_mean, bn1_var, w2, b2):
    col = lambda p: p.reshape(-1, 1)
    wspec = lambda s: pl.BlockSpec(s, lambda b: (0,) * len(s))
    return pl.pallas_call(
        _head_body,
        grid=(B,),
        in_specs=[
            pl.BlockSpec((1, C, NS), lambda b: (b, 0, 0)),
            wspec((C, C)), wspec((C, 1)), wspec((C, 1)), wspec((C, 1)),
            wspec((C, 1)), wspec((C, 1)), wspec((3, C)), wspec((3, 1)),
            wspec((300, 3)),
        ],
        out_specs=[
            pl.BlockSpec((1, 3, NS), lambda b: (b, 0, 0)),
            pl.BlockSpec((1, 1, NS), lambda b: (b, 0, 0)),
            pl.BlockSpec((1, 9, NS), lambda b: (b, 0, 0)),
        ],
        out_shape=[
            jax.ShapeDtypeStruct((B, 3, NS), jnp.float32),
            jax.ShapeDtypeStruct((B, 1, NS), jnp.int32),
            jax.ShapeDtypeStruct((B, 9, NS), jnp.float32),
        ],
    )(gf, w1, col(b1), col(bn1_gamma), col(bn1_beta), col(bn1_mean),
      col(bn1_var), w2, col(b2), _TMPL)


@jax.jit
def kernel(seed_xyz, seed_features, w_g1, b_g1, g_gamma, g_beta, g_mean,
           g_var, w_g2, b_g2, w1, b1, bn1_gamma, bn1_beta, bn1_mean,
           bn1_var, w2, b2):
    g3, mask, g2r = _scores(seed_features, w_g1, b_g1, g_gamma, g_beta,
                            g_mean, g_var, w_g2, b_g2)
    objectness_score = g3[:, :2]
    graspness_score = g3[:, 2]

    def chunks(a_bn):
        ap = jnp.pad(a_bn, ((0, 0), (0, NPAD - N))).reshape(B, NCH, 8, 128)
        return jnp.transpose(ap, (1, 0, 2, 3)).reshape(NCH, B * 8, 128)

    inds8, gx, gy, gz, gp = _fps(
        chunks(seed_xyz[:, :, 0]), chunks(seed_xyz[:, :, 1]),
        chunks(seed_xyz[:, :, 2]), chunks(g2r.reshape(B, N)),
        chunks(mask.reshape(B, N)))
    graspable_inds = inds8.reshape(B, NS)
    graspable_xyz = jnp.stack(
        [gx.reshape(B, NS), gy.reshape(B, NS), gz.reshape(B, NS)], axis=-1)
    fp2_graspness = gp.reshape(B, NS)

    graspable_features = _gather(seed_features, graspable_inds)

    v, tvi, rot = _head(graspable_features, w1, b1, bn1_gamma, bn1_beta,
                        bn1_mean, bn1_var, w2, b2)
    vp_xyz = jnp.transpose(v, (0, 2, 1))
    top_view_inds = tvi.reshape(B, NS)
    vp_rot = jnp.transpose(rot, (0, 2, 1)).reshape(B, NS, 3, 3)

    return (objectness_score, graspness_score, graspable_xyz,
            graspable_inds, graspable_features, fp2_graspness, vp_xyz,
            top_view_inds, vp_rot)
